# Initial kernel scaffold; baseline (speedup 1.0000x reference)
import jax
import jax.numpy as jnp
from jax.experimental import pallas as pl


def kernel(density, z_vals, rays_d, u, N_importance):
    raise NotImplementedError("write your pallas kernel here")


# speedup vs baseline: 854.0717x; 854.0717x over previous
import functools

import jax
import jax.numpy as jnp
import numpy as np
from jax.experimental import pallas as pl
from jax.experimental.pallas import tpu as pltpu

S = 64
NI = 128
R = 256


def _roll_left(x, s):
    if s == 0:
        return x
    return jnp.concatenate([x[:, s:], x[:, :s]], axis=1)


def _bitonic_sort_lanes(z, lane):
    n = z.shape[1]
    k = 2
    while k <= n:
        j = k // 2
        while j >= 1:
            partner = jnp.where((lane & j) == 0, _roll_left(z, j), _roll_left(z, n - j))
            lo = jnp.minimum(z, partner)
            hi = jnp.maximum(z, partner)
            take_lo = ((lane & j) == 0) ^ ((lane & k) != 0)
            z = jnp.where(take_lo, lo, hi)
            j //= 2
        k *= 2
    return z


def _sample_kernel(density_ref, z_ref, rays_ref, u_ref, l64_ref, l62_ref, out_ref):
    dens = density_ref[...]
    z_vals = z_ref[...]
    rays = rays_ref[...]
    u = u_ref[...]
    lower64 = l64_ref[...]
    lower62 = l62_ref[...]

    norm = jnp.sqrt(jnp.sum(rays * rays, axis=1, keepdims=True))

    diffs = (z_vals[:, 1:] - z_vals[:, :-1]) * 100.0
    dists = jnp.concatenate(
        [diffs, jnp.full((diffs.shape[0], 1), 1e10, dtype=diffs.dtype)], axis=1)
    dists = dists * norm

    x = jnp.maximum(dens, 0.0) * dists
    t = jnp.exp(-x)
    alpha = 1.0 - t
    logt = jnp.log(t + 1e-10)
    log_trans = jax.lax.dot_general(
        logt, lower64, (((1,), (0,)), ((), ())),
        precision=jax.lax.Precision.HIGHEST,
        preferred_element_type=jnp.float32)
    weights = alpha * jnp.exp(log_trans)

    w = weights[:, 1:S - 1] + 1e-5
    total = jnp.sum(w, axis=1, keepdims=True)
    cdf_un = jax.lax.dot_general(
        w, lower62, (((1,), (0,)), ((), ())),
        precision=jax.lax.Precision.HIGHEST,
        preferred_element_type=jnp.float32)
    cdf = cdf_un / total

    bins = 0.5 * (z_vals[:, 1:] + z_vals[:, :-1])

    denom = cdf[:, 1:] - cdf[:, :-1]
    denom = jnp.where(denom < 1e-5, 1.0, denom)
    slope = (bins[:, 1:] - bins[:, :-1]) / denom
    slope = jnp.concatenate(
        [slope, jnp.zeros((slope.shape[0], 1), dtype=slope.dtype)], axis=1)
    intercept = bins - cdf * slope

    z = intercept[:, 0:1] + u * slope[:, 0:1]
    for j in range(1, S - 1):
        mask = u >= cdf[:, j:j + 1]
        z = jnp.where(mask, intercept[:, j:j + 1] + u * slope[:, j:j + 1], z)

    lane = jax.lax.broadcasted_iota(jnp.int32, (1, NI), 1)
    out_ref[...] = _bitonic_sort_lanes(z, lane)


@jax.jit
def _run(density2, z_vals, rays_d, u):
    n = density2.shape[0]
    lower64 = jnp.asarray(np.triu(np.ones((S, S), np.float32), 1))
    lower62 = jnp.asarray(np.triu(np.ones((S - 2, S - 1), np.float32), 1))
    grid = (n // R,)
    return pl.pallas_call(
        _sample_kernel,
        grid=grid,
        in_specs=[
            pl.BlockSpec((R, S), lambda i: (i, 0)),
            pl.BlockSpec((R, S), lambda i: (i, 0)),
            pl.BlockSpec((R, 3), lambda i: (i, 0)),
            pl.BlockSpec((R, NI), lambda i: (i, 0)),
            pl.BlockSpec((S, S), lambda i: (0, 0)),
            pl.BlockSpec((S - 2, S - 1), lambda i: (0, 0)),
        ],
        out_specs=pl.BlockSpec((R, NI), lambda i: (i, 0)),
        out_shape=jax.ShapeDtypeStruct((n, NI), jnp.float32),
    )(density2, z_vals, rays_d, u, lower64, lower62)


def kernel(density, z_vals, rays_d, u, N_importance):
    del N_importance
    return _run(density[..., 0], z_vals, rays_d, u)


# speedup vs baseline: 856.0214x; 1.0023x over previous
import functools

import jax
import jax.numpy as jnp
import numpy as np
from jax.experimental import pallas as pl
from jax.experimental.pallas import tpu as pltpu

S = 64
NI = 128
R = 256


def _roll_left(x, s):
    if s == 0:
        return x
    return jnp.concatenate([x[:, s:], x[:, :s]], axis=1)


def _bitonic_sort_lanes(z, lane):
    n = z.shape[1]
    k = 2
    while k <= n:
        j = k // 2
        while j >= 1:
            partner = jnp.where((lane & j) == 0, _roll_left(z, j), _roll_left(z, n - j))
            lo = jnp.minimum(z, partner)
            hi = jnp.maximum(z, partner)
            take_lo = ((lane & j) == 0) ^ ((lane & k) != 0)
            z = jnp.where(take_lo, lo, hi)
            j //= 2
        k *= 2
    return z


def _sample_kernel(density_ref, z_ref, rays_ref, u_ref, l64_ref, l62_ref, out_ref):
    dens = density_ref[...]
    z_vals = z_ref[...]
    rays = rays_ref[...]
    u = u_ref[...]
    lower64 = l64_ref[...]
    lower62 = l62_ref[...]

    norm = jnp.sqrt(jnp.sum(rays * rays, axis=1, keepdims=True))

    diffs = (z_vals[:, 1:] - z_vals[:, :-1]) * 100.0
    dists = jnp.concatenate(
        [diffs, jnp.full((diffs.shape[0], 1), 1e10, dtype=diffs.dtype)], axis=1)
    dists = dists * norm

    x = jnp.maximum(dens, 0.0) * dists
    cx = jax.lax.dot_general(
        x, lower64, (((1,), (0,)), ((), ())),
        precision=jax.lax.Precision.HIGHEST,
        preferred_element_type=jnp.float32)
    weights = jnp.exp(-cx) - jnp.exp(-(cx + x))

    w = weights[:, 1:S - 1] + 1e-5
    total = jnp.sum(w, axis=1, keepdims=True)
    cdf_un = jax.lax.dot_general(
        w, lower62, (((1,), (0,)), ((), ())),
        precision=jax.lax.Precision.HIGHEST,
        preferred_element_type=jnp.float32)
    cdf = cdf_un / total

    bins = 0.5 * (z_vals[:, 1:] + z_vals[:, :-1])

    denom = cdf[:, 1:] - cdf[:, :-1]
    denom = jnp.where(denom < 1e-5, 1.0, denom)
    slope = (bins[:, 1:] - bins[:, :-1]) / denom
    slope = jnp.concatenate(
        [slope, jnp.zeros((slope.shape[0], 1), dtype=slope.dtype)], axis=1)
    intercept = bins - cdf * slope

    z = intercept[:, 0:1] + u * slope[:, 0:1]
    for j in range(1, S - 1):
        mask = u >= cdf[:, j:j + 1]
        z = jnp.where(mask, intercept[:, j:j + 1] + u * slope[:, j:j + 1], z)

    lane = jax.lax.broadcasted_iota(jnp.int32, (1, NI), 1)
    out_ref[...] = _bitonic_sort_lanes(z, lane)


@jax.jit
def _run(density2, z_vals, rays_d, u):
    n = density2.shape[0]
    lower64 = jnp.asarray(np.triu(np.ones((S, S), np.float32), 1))
    lower62 = jnp.asarray(np.triu(np.ones((S - 2, S - 1), np.float32), 1))
    grid = (n // R,)
    return pl.pallas_call(
        _sample_kernel,
        grid=grid,
        in_specs=[
            pl.BlockSpec((R, S), lambda i: (i, 0)),
            pl.BlockSpec((R, S), lambda i: (i, 0)),
            pl.BlockSpec((R, 3), lambda i: (i, 0)),
            pl.BlockSpec((R, NI), lambda i: (i, 0)),
            pl.BlockSpec((S, S), lambda i: (0, 0)),
            pl.BlockSpec((S - 2, S - 1), lambda i: (0, 0)),
        ],
        out_specs=pl.BlockSpec((R, NI), lambda i: (i, 0)),
        out_shape=jax.ShapeDtypeStruct((n, NI), jnp.float32),
    )(density2, z_vals, rays_d, u, lower64, lower62)


def kernel(density, z_vals, rays_d, u, N_importance):
    del N_importance
    return _run(density[..., 0], z_vals, rays_d, u)


# speedup vs baseline: 3629.2261x; 4.2396x over previous
import jax
import jax.numpy as jnp
import numpy as np
from jax.experimental import pallas as pl

S = 64
NI = 128
RB = 512
NROW = NI // 8


def _xor_shuffle8(c, j):
    if j == 4:
        return jnp.concatenate([c[4:8], c[0:4]], axis=0)
    if j == 2:
        return jnp.concatenate([c[2:4], c[0:2], c[6:8], c[4:6]], axis=0)
    return jnp.concatenate(
        [c[1:2], c[0:1], c[3:4], c[2:3], c[5:6], c[4:5], c[7:8], c[6:7]], axis=0)


def _sort128_sublanes(zs):
    rb = zs[0].shape[1]
    iota8 = jax.lax.broadcasted_iota(jnp.int32, (8, rb), 0)
    pat = {j: (iota8 & j) == 0 for j in (1, 2, 4)}
    mk = {(k, j): ((iota8 & j) == 0) ^ ((iota8 & k) != 0)
          for (k, j) in ((2, 1), (4, 2), (4, 1))}

    k = 2
    while k <= NI:
        j = k // 2
        while j >= 1:
            if j >= 8:
                jr = j // 8
                new = []
                for r in range(NROW):
                    a, b = zs[r], zs[r ^ jr]
                    take_lo = (((r * 8) & j) == 0) ^ (((r * 8) & k) != 0)
                    new.append(jnp.minimum(a, b) if take_lo else jnp.maximum(a, b))
                zs = new
            else:
                mask = mk[(k, j)] if k <= 4 else pat[j]
                for r in range(NROW):
                    c = zs[r]
                    p = _xor_shuffle8(c, j)
                    lo = jnp.minimum(c, p)
                    hi = jnp.maximum(c, p)
                    flip = ((r * 8) & k) != 0 if k >= 8 else False
                    zs[r] = jnp.where(mask, hi, lo) if flip else jnp.where(mask, lo, hi)
            j //= 2
        k *= 2
    return zs


def _sample_kernel(density_ref, z_ref, rays_ref, u_ref, m64_ref, m63_ref, out_ref):
    dens = density_ref[...]
    z_vals = z_ref[...]
    rays = rays_ref[...]
    m64 = m64_ref[...]
    m63 = m63_ref[...]

    norm = jnp.sqrt(jnp.sum(rays * rays, axis=0, keepdims=True))

    diffs = (z_vals[1:] - z_vals[:-1]) * 100.0
    dists = jnp.concatenate(
        [diffs, jnp.full((1, diffs.shape[1]), 1e10, dtype=diffs.dtype)], axis=0)
    dists = dists * norm

    x = jnp.maximum(dens, 0.0) * dists
    cx = jax.lax.dot_general(
        m64, x, (((1,), (0,)), ((), ())),
        precision=jax.lax.Precision.HIGHEST,
        preferred_element_type=jnp.float32)
    weights = jnp.exp(-cx) - jnp.exp(-(cx + x))

    w = weights[1:S - 1] + 1e-5
    tot = jnp.sum(w, axis=0, keepdims=True)
    cdf = jax.lax.dot_general(
        m63, w, (((1,), (0,)), ((), ())),
        precision=jax.lax.Precision.HIGHEST,
        preferred_element_type=jnp.float32) / tot

    bins = 0.5 * (z_vals[1:] + z_vals[:-1])

    denom = cdf[1:] - cdf[:-1]
    denom = jnp.where(denom < 1e-5, 1.0, denom)
    slope = (bins[1:] - bins[:-1]) / denom
    slope = jnp.concatenate(
        [slope, jnp.zeros((1, slope.shape[1]), dtype=slope.dtype)], axis=0)
    intercept = bins - cdf * slope

    u_all = u_ref[...]
    us = [u_all[8 * r:8 * r + 8] for r in range(NROW)]

    ab = jnp.broadcast_to(intercept[0:1], (8, RB))
    sb = jnp.broadcast_to(slope[0:1], (8, RB))
    zs = [ab + us[r] * sb for r in range(NROW)]
    for j in range(1, S - 1):
        cb = jnp.broadcast_to(cdf[j:j + 1], (8, RB))
        ab = jnp.broadcast_to(intercept[j:j + 1], (8, RB))
        sb = jnp.broadcast_to(slope[j:j + 1], (8, RB))
        for r in range(NROW):
            zs[r] = jnp.where(us[r] >= cb, ab + us[r] * sb, zs[r])

    zs = _sort128_sublanes(zs)
    out_ref[...] = jnp.concatenate(zs, axis=0)


@jax.jit
def _run(density_t, z_t, rays_t, u_t):
    n = density_t.shape[1]
    m64 = jnp.asarray(np.tril(np.ones((S, S), np.float32), -1))
    m63 = jnp.asarray(np.tril(np.ones((S - 1, S - 2), np.float32), -1))
    out_t = pl.pallas_call(
        _sample_kernel,
        grid=(n // RB,),
        in_specs=[
            pl.BlockSpec((S, RB), lambda i: (0, i)),
            pl.BlockSpec((S, RB), lambda i: (0, i)),
            pl.BlockSpec((8, RB), lambda i: (0, i)),
            pl.BlockSpec((NI, RB), lambda i: (0, i)),
            pl.BlockSpec((S, S), lambda i: (0, 0)),
            pl.BlockSpec((S - 1, S - 2), lambda i: (0, 0)),
        ],
        out_specs=pl.BlockSpec((NI, RB), lambda i: (0, i)),
        out_shape=jax.ShapeDtypeStruct((NI, n), jnp.float32),
    )(density_t, z_t, rays_t, u_t, m64, m63)
    return out_t


def kernel(density, z_vals, rays_d, u, N_importance):
    del N_importance
    n = density.shape[0]
    rays_pad = jnp.concatenate(
        [rays_d, jnp.zeros((n, 5), rays_d.dtype)], axis=1)
    out_t = _run(density[..., 0].T, z_vals.T, rays_pad.T, u.T)
    return out_t.T
